# Initial kernel scaffold; baseline (speedup 1.0000x reference)
import jax
import jax.numpy as jnp
from jax.experimental import pallas as pl


def kernel(x, eid, stitch_W, stitch_b, proj_W, proj_b):
    raise NotImplementedError("write your pallas kernel here")


# speedup vs baseline: 1.2535x; 1.2535x over previous
import jax
import jax.numpy as jnp
from jax.experimental import pallas as pl
from jax.experimental.pallas import tpu as pltpu


def _stitch_kernel(idx_ref, x_ref, sW_ref, sb_ref, pW_ref, pb_ref, o_ref):
    x = x_ref[0]
    h = jnp.dot(x, sW_ref[0], preferred_element_type=jnp.float32)
    h = h + sb_ref[0]
    h = h / (1.0 + jnp.abs(h))
    o = jnp.dot(h, pW_ref[0], preferred_element_type=jnp.float32)
    o_ref[0] = o + pb_ref[0]


def kernel(x, eid, stitch_W, stitch_b, proj_W, proj_b):
    B, F, N = x.shape
    E, _, M = stitch_W.shape
    P = proj_W.shape[-1]

    eid32 = eid.astype(jnp.int32)
    order = jnp.argsort(eid32).astype(jnp.int32)
    eid_sorted = eid32[order]
    idx = jnp.stack([order, eid_sorted])

    sb3 = stitch_b.reshape(E, 1, M)
    pb3 = proj_b.reshape(E, 1, P)

    grid_spec = pltpu.PrefetchScalarGridSpec(
        num_scalar_prefetch=1,
        grid=(B,),
        in_specs=[
            pl.BlockSpec((1, F, N), lambda i, idx: (idx[0, i], 0, 0)),
            pl.BlockSpec((1, N, M), lambda i, idx: (idx[1, i], 0, 0)),
            pl.BlockSpec((1, 1, M), lambda i, idx: (idx[1, i], 0, 0)),
            pl.BlockSpec((1, M, P), lambda i, idx: (idx[1, i], 0, 0)),
            pl.BlockSpec((1, 1, P), lambda i, idx: (idx[1, i], 0, 0)),
        ],
        out_specs=pl.BlockSpec((1, F, P), lambda i, idx: (idx[0, i], 0, 0)),
    )
    return pl.pallas_call(
        _stitch_kernel,
        grid_spec=grid_spec,
        out_shape=jax.ShapeDtypeStruct((B, F, P), jnp.float32),
    )(idx, x, stitch_W, sb3, proj_W, pb3)


# speedup vs baseline: 1.2574x; 1.0031x over previous
import jax
import jax.numpy as jnp
from jax.experimental import pallas as pl
from jax.experimental.pallas import tpu as pltpu


def _stitch_kernel(idx_ref, x_ref, sW_ref, sb_ref, pW_ref, pb_ref, o_ref):
    x = x_ref[0].astype(jnp.bfloat16)
    h = jnp.dot(x, sW_ref[0].astype(jnp.bfloat16),
                preferred_element_type=jnp.float32)
    h = h + sb_ref[0]
    h = h / (1.0 + jnp.abs(h))
    o = jnp.dot(h.astype(jnp.bfloat16), pW_ref[0].astype(jnp.bfloat16),
                preferred_element_type=jnp.float32)
    o_ref[0] = o + pb_ref[0]


def kernel(x, eid, stitch_W, stitch_b, proj_W, proj_b):
    B, F, N = x.shape
    E, _, M = stitch_W.shape
    P = proj_W.shape[-1]

    eid32 = eid.astype(jnp.int32)
    order = jnp.argsort(eid32).astype(jnp.int32)
    eid_sorted = eid32[order]
    idx = jnp.stack([order, eid_sorted])

    sb3 = stitch_b.reshape(E, 1, M)
    pb3 = proj_b.reshape(E, 1, P)

    grid_spec = pltpu.PrefetchScalarGridSpec(
        num_scalar_prefetch=1,
        grid=(B,),
        in_specs=[
            pl.BlockSpec((1, F, N), lambda i, idx: (idx[0, i], 0, 0)),
            pl.BlockSpec((1, N, M), lambda i, idx: (idx[1, i], 0, 0)),
            pl.BlockSpec((1, 1, M), lambda i, idx: (idx[1, i], 0, 0)),
            pl.BlockSpec((1, M, P), lambda i, idx: (idx[1, i], 0, 0)),
            pl.BlockSpec((1, 1, P), lambda i, idx: (idx[1, i], 0, 0)),
        ],
        out_specs=pl.BlockSpec((1, F, P), lambda i, idx: (idx[0, i], 0, 0)),
    )
    return pl.pallas_call(
        _stitch_kernel,
        grid_spec=grid_spec,
        out_shape=jax.ShapeDtypeStruct((B, F, P), jnp.float32),
    )(idx, x, stitch_W, sb3, proj_W, pb3)


# speedup vs baseline: 1.3404x; 1.0660x over previous
import jax
import jax.numpy as jnp
from jax.experimental import pallas as pl
from jax.experimental.pallas import tpu as pltpu


def _stitch_kernel(eid_ref, x_ref, sW_ref, sb_ref, pW_ref, pb_ref, o_ref):
    e = eid_ref[pl.program_id(0)]
    x = x_ref[0]
    h = jnp.dot(x, sW_ref[e], preferred_element_type=jnp.float32)
    h = h + sb_ref[e]
    h = h / (1.0 + jnp.abs(h))
    o = jnp.dot(h, pW_ref[e], preferred_element_type=jnp.float32)
    o_ref[0] = o + pb_ref[e]


def kernel(x, eid, stitch_W, stitch_b, proj_W, proj_b):
    B, F, N = x.shape
    E, _, M = stitch_W.shape
    P = proj_W.shape[-1]

    eid32 = eid.astype(jnp.int32)
    sb3 = stitch_b.reshape(E, 1, M)
    pb3 = proj_b.reshape(E, 1, P)

    grid_spec = pltpu.PrefetchScalarGridSpec(
        num_scalar_prefetch=1,
        grid=(B,),
        in_specs=[
            pl.BlockSpec((1, F, N), lambda i, eid: (i, 0, 0)),
            pl.BlockSpec((E, N, M), lambda i, eid: (0, 0, 0)),
            pl.BlockSpec((E, 1, M), lambda i, eid: (0, 0, 0)),
            pl.BlockSpec((E, M, P), lambda i, eid: (0, 0, 0)),
            pl.BlockSpec((E, 1, P), lambda i, eid: (0, 0, 0)),
        ],
        out_specs=pl.BlockSpec((1, F, P), lambda i, eid: (i, 0, 0)),
    )
    return pl.pallas_call(
        _stitch_kernel,
        grid_spec=grid_spec,
        out_shape=jax.ShapeDtypeStruct((B, F, P), jnp.float32),
    )(eid32, x, stitch_W, sb3, proj_W, pb3)


# speedup vs baseline: 1.3447x; 1.0032x over previous
import jax
import jax.numpy as jnp
from jax.experimental import pallas as pl
from jax.experimental.pallas import tpu as pltpu


def _stitch_kernel(eid_ref, x_ref, sW_ref, sb_ref, pW_ref, pb_ref, o_ref):
    e = eid_ref[pl.program_id(0)]
    x = x_ref[0]
    h = jnp.dot(x, sW_ref[e], preferred_element_type=jnp.float32)
    h = h + sb_ref[e]
    h = h / (1.0 + jnp.abs(h))
    o = jnp.dot(h, pW_ref[e], preferred_element_type=jnp.float32)
    o_ref[0] = o + pb_ref[e]


def kernel(x, eid, stitch_W, stitch_b, proj_W, proj_b):
    B, F, N = x.shape
    E, _, M = stitch_W.shape
    P = proj_W.shape[-1]

    eid32 = eid.astype(jnp.int32)
    sb3 = stitch_b.reshape(E, 1, M)
    pb3 = proj_b.reshape(E, 1, P)

    grid_spec = pltpu.PrefetchScalarGridSpec(
        num_scalar_prefetch=1,
        grid=(B,),
        in_specs=[
            pl.BlockSpec((1, F, N), lambda i, eid: (i, 0, 0)),
            pl.BlockSpec((E, N, M), lambda i, eid: (0, 0, 0)),
            pl.BlockSpec((E, 1, M), lambda i, eid: (0, 0, 0)),
            pl.BlockSpec((E, M, P), lambda i, eid: (0, 0, 0)),
            pl.BlockSpec((E, 1, P), lambda i, eid: (0, 0, 0)),
        ],
        out_specs=pl.BlockSpec((1, F, P), lambda i, eid: (i, 0, 0)),
    )
    return pl.pallas_call(
        _stitch_kernel,
        grid_spec=grid_spec,
        out_shape=jax.ShapeDtypeStruct((B, F, P), jnp.float32),
        compiler_params=pltpu.CompilerParams(
            dimension_semantics=("parallel",),
        ),
    )(eid32, x, stitch_W, sb3, proj_W, pb3)
